# Initial kernel scaffold; baseline (speedup 1.0000x reference)
import jax
import jax.numpy as jnp
from jax.experimental import pallas as pl


def kernel(img):
    raise NotImplementedError("write your pallas kernel here")


# speedup vs baseline: 28.8022x; 28.8022x over previous
import jax
import jax.numpy as jnp
from jax.experimental import pallas as pl
from jax.experimental.pallas import tpu as pltpu

_B = 64
_BK = _B // 2


def _body(in_hbm, out_ref, scratch, sem):
    i = pl.program_id(0)
    j = pl.program_id(1)
    cp = pltpu.make_async_copy(
        in_hbm.at[i, pl.ds(j * _BK, _BK + 2)], scratch, sem
    )
    cp.start()
    cp.wait()
    s = scratch[...]
    a = s[0:_BK]
    b = s[1:_BK + 1]
    c = s[2:_BK + 2]
    even = 0.25 * a + 0.75 * b
    odd = 0.75 * b + 0.25 * c
    rows = jnp.stack([even, odd], axis=1).reshape(_B, 226, 64)
    ex = 0.25 * rows[:, 0:224] + 0.75 * rows[:, 1:225]
    ox = 0.75 * rows[:, 1:225] + 0.25 * rows[:, 2:226]
    out_ref[0] = jnp.stack([ex, ox], axis=2).reshape(_B, 448, 64)


def kernel(img):
    n, h, w, ch = img.shape
    pad = jnp.pad(img, ((0, 0), (1, 1), (1, 1), (0, 0)), mode="edge")
    return pl.pallas_call(
        _body,
        grid=(n, 2 * h // _B),
        in_specs=[pl.BlockSpec(memory_space=pltpu.MemorySpace.HBM)],
        out_specs=pl.BlockSpec((1, _B, 2 * w, ch), lambda i, j: (i, j, 0, 0)),
        out_shape=jax.ShapeDtypeStruct((n, 2 * h, 2 * w, ch), img.dtype),
        scratch_shapes=[
            pltpu.VMEM((_BK + 2, w + 2, ch), jnp.float32),
            pltpu.SemaphoreType.DMA,
        ],
    )(pad)


# speedup vs baseline: 38.1075x; 1.3231x over previous
import jax
import jax.numpy as jnp
from jax import lax
from jax.experimental import pallas as pl
from jax.experimental.pallas import tpu as pltpu
from jax.experimental.pallas import tpu_sc as plsc

_H = 224
_ROW = 224 * 64
_OROW = 2 * _ROW
_HALF = _ROW // 2
_OHALF = _ROW
_PX = 64
_NPAIR = 14
_T = _HALF // _PX
_HB = _HALF + 2 * _PX
_DMA = _HALF + _PX


def _inb_base(b, j):
    return (b * 3 + j) * _HB


def _outb_base(b, r):
    return (b * 2 + r) * _OHALF


def _body(img, out, inb, outb, in_s0, in_s1, out_s0, out_s1):
    batch = lax.axis_index("c")
    seg = lax.axis_index("s")
    k0 = seg * _NPAIR
    in_sems = (in_s0, in_s1)
    out_sems = (out_s0, out_s1)

    def issue_in(b, h, k):
        rp = jnp.maximum(k - 1, 0)
        rn = jnp.minimum(k + 1, _H - 1)
        src0 = 0 if h == 0 else _HALF - _PX
        dst0 = _PX if h == 0 else 0
        for j, r in enumerate((rp, k, rn)):
            off = pl.multiple_of((batch * _H + r) * _ROW + src0, _PX)
            pltpu.async_copy(
                img.at[pl.ds(off, _DMA)],
                inb.at[pl.ds(_inb_base(b, j) + dst0, _DMA)],
                in_sems[b],
            )

    def wait_in(b, h):
        src0 = 0 if h == 0 else _HALF - _PX
        dst0 = _PX if h == 0 else 0
        for j in range(3):
            pltpu.make_async_copy(
                img.at[pl.ds(src0, _DMA)],
                inb.at[pl.ds(_inb_base(b, j) + dst0, _DMA)],
                in_sems[b],
            ).wait()
        lo = 0 if h == 0 else _HB - _PX
        src = _PX if h == 0 else _HB - 2 * _PX
        for j in range(3):
            base = _inb_base(b, j)
            for c in range(0, _PX, 16):
                inb[pl.ds(base + lo + c, 16)] = inb[pl.ds(base + src + c, 16)]

    def issue_out(b, h, k):
        for r in range(2):
            off = pl.multiple_of(
                (batch * 2 * _H + 2 * k + r) * _OROW + h * _OHALF, _PX
            )
            pltpu.async_copy(
                outb.at[pl.ds(_outb_base(b, r), _OHALF)],
                out.at[pl.ds(off, _OHALF)],
                out_sems[b],
            )

    def wait_out(b, h):
        for r in range(2):
            pltpu.make_async_copy(
                outb.at[pl.ds(_outb_base(b, r), _OHALF)],
                out.at[pl.ds(h * _OHALF, _OHALF)],
                out_sems[b],
            ).wait()

    def compute(b, h):
        pb = _inb_base(b, 0)
        cb = _inb_base(b, 1)
        nb = _inb_base(b, 2)
        eb = _outb_base(b, 0)
        ob = _outb_base(b, 1)

        def ey_oy(p_off):
            ey, oy = [], []
            for c in range(0, _PX, 16):
                pc = inb[pl.ds(pb + p_off + c, 16)]
                cc = inb[pl.ds(cb + p_off + c, 16)]
                nc = inb[pl.ds(nb + p_off + c, 16)]
                ey.append(0.25 * pc + 0.75 * cc)
                oy.append(0.75 * cc + 0.25 * nc)
            return ey, oy

        ey_m1, oy_m1 = ey_oy(0)
        ey_0, oy_0 = ey_oy(_PX)

        def step(t, carry):
            ey_m1 = carry[0:4]
            ey_0 = carry[4:8]
            oy_m1 = carry[8:12]
            oy_0 = carry[12:16]
            p_off = pl.multiple_of((t + 2) * _PX, _PX)
            ey_p1, oy_p1 = ey_oy(p_off)
            sb = pl.multiple_of(2 * _PX * t, 2 * _PX)
            for c in range(4):
                outb[pl.ds(eb + sb + 16 * c, 16)] = (
                    0.25 * ey_m1[c] + 0.75 * ey_0[c])
                outb[pl.ds(eb + sb + _PX + 16 * c, 16)] = (
                    0.75 * ey_0[c] + 0.25 * ey_p1[c])
                outb[pl.ds(ob + sb + 16 * c, 16)] = (
                    0.25 * oy_m1[c] + 0.75 * oy_0[c])
                outb[pl.ds(ob + sb + _PX + 16 * c, 16)] = (
                    0.75 * oy_0[c] + 0.25 * oy_p1[c])
            return tuple(ey_0) + tuple(ey_p1) + tuple(oy_0) + tuple(oy_p1)

        lax.fori_loop(
            0, _T, step,
            tuple(ey_m1) + tuple(ey_0) + tuple(oy_m1) + tuple(oy_0),
            unroll=2,
        )

    def do_pair(b, h, k, prefetch_k, do_wait_out):
        wait_in(b, h)
        if prefetch_k is not None:
            issue_in(1 - b, h, prefetch_k)
        if do_wait_out:
            wait_out(b, h)
        compute(b, h)
        issue_out(b, h, k)

    for h in range(2):
        issue_in(0, h, k0)
        do_pair(0, h, k0, k0 + 1, False)
        do_pair(1, h, k0 + 1, k0 + 2, False)

        def loop(i, _):
            do_pair(0, h, k0 + 2 * i, k0 + 2 * i + 1, True)
            do_pair(1, h, k0 + 2 * i + 1, k0 + 2 * i + 2, True)
            return 0

        lax.fori_loop(1, _NPAIR // 2 - 1, loop, 0)
        do_pair(0, h, k0 + _NPAIR - 2, k0 + _NPAIR - 1, True)
        do_pair(1, h, k0 + _NPAIR - 1, None, True)
        wait_out(0, h)
        wait_out(1, h)


def kernel(img):
    n, h, w, ch = img.shape
    flat = img.reshape(n * h * w * ch)
    run = pl.kernel(
        _body,
        out_type=jax.ShapeDtypeStruct((n * 2 * h * 2 * w * ch,), jnp.float32),
        mesh=plsc.VectorSubcoreMesh(
            core_axis_name="c", subcore_axis_name="s",
            num_cores=2, num_subcores=16,
        ),
        scratch_types=[
            pltpu.VMEM((2 * 3 * _HB,), jnp.float32),
            pltpu.VMEM((2 * 2 * _OHALF,), jnp.float32),
            pltpu.SemaphoreType.DMA,
            pltpu.SemaphoreType.DMA,
            pltpu.SemaphoreType.DMA,
            pltpu.SemaphoreType.DMA,
        ],
    )
    out_flat = run(flat)
    return out_flat.reshape(n, 2 * h, 2 * w, ch)


# speedup vs baseline: 43.3711x; 1.1381x over previous
import jax
import jax.numpy as jnp
from jax import lax
from jax.experimental import pallas as pl
from jax.experimental.pallas import tpu as pltpu
from jax.experimental.pallas import tpu_sc as plsc

_H = 224
_W = 224
_C = 64
_NPAIR = 14
_QW = 56
_T = _QW
_IB = 72


def _body(img, out, inb, outb, in_s0, in_s1, out_s0, out_s1):
    batch = lax.axis_index("c")
    seg = lax.axis_index("s")
    k0 = seg * _NPAIR
    in_sems = (in_s0, in_s1)
    out_sems = (out_s0, out_s1)

    def src_window(q):
        w0 = q * _QW
        src_lo = max(w0 - 8, 0)
        src_hi = min(w0 + _QW + 8, _W)
        return w0, src_lo, src_hi - src_lo

    def issue_in(b, q, k):
        rp = jnp.maximum(k - 1, 0)
        rn = jnp.minimum(k + 1, _H - 1)
        w0, src_lo, n = src_window(q)
        dst_lo = src_lo - w0 + 8
        for j, r in enumerate((rp, k, rn)):
            pltpu.async_copy(
                img.at[batch, r, pl.ds(src_lo, n), :],
                inb.at[b, j, pl.ds(dst_lo, n), :],
                in_sems[b],
            )

    def wait_in(b, q):
        w0, src_lo, n = src_window(q)
        dst_lo = src_lo - w0 + 8
        for j in range(3):
            pltpu.make_async_copy(
                img.at[0, 0, pl.ds(src_lo, n), :],
                inb.at[b, j, pl.ds(dst_lo, n), :],
                in_sems[b],
            ).wait()
        if q == 0:
            dst_px, src_px = 7, 8
        elif q == 3:
            dst_px, src_px = _QW + 8, _QW + 7
        else:
            return
        for j in range(3):
            for c in range(0, _C, 16):
                inb[b, j, dst_px, pl.ds(c, 16)] = inb[b, j, src_px, pl.ds(c, 16)]

    def issue_out(b, q, k):
        for r in range(2):
            pltpu.async_copy(
                outb.at[b, r],
                out.at[batch, 2 * k + r, pl.ds(q * 2 * _QW, 2 * _QW), :],
                out_sems[b],
            )

    def wait_out(b, q):
        for r in range(2):
            pltpu.make_async_copy(
                outb.at[b, r],
                out.at[0, 0, pl.ds(q * 2 * _QW, 2 * _QW), :],
                out_sems[b],
            ).wait()

    def compute(b):
        def ey_oy(slot):
            ey, oy = [], []
            for c in range(0, _C, 16):
                pc = inb[b, 0, slot, pl.ds(c, 16)]
                cc = inb[b, 1, slot, pl.ds(c, 16)]
                nc = inb[b, 2, slot, pl.ds(c, 16)]
                ey.append(0.25 * pc + 0.75 * cc)
                oy.append(0.75 * cc + 0.25 * nc)
            return ey, oy

        ey_m1, oy_m1 = ey_oy(7)
        ey_0, oy_0 = ey_oy(8)

        def step(t, carry):
            ey_m1 = carry[0:4]
            ey_0 = carry[4:8]
            oy_m1 = carry[8:12]
            oy_0 = carry[12:16]
            ey_p1, oy_p1 = ey_oy(t + 9)
            for c in range(4):
                cs = pl.ds(16 * c, 16)
                outb[b, 0, 2 * t, cs] = 0.25 * ey_m1[c] + 0.75 * ey_0[c]
                outb[b, 0, 2 * t + 1, cs] = 0.75 * ey_0[c] + 0.25 * ey_p1[c]
                outb[b, 1, 2 * t, cs] = 0.25 * oy_m1[c] + 0.75 * oy_0[c]
                outb[b, 1, 2 * t + 1, cs] = 0.75 * oy_0[c] + 0.25 * oy_p1[c]
            return tuple(ey_0) + tuple(ey_p1) + tuple(oy_0) + tuple(oy_p1)

        lax.fori_loop(
            0, _T, step,
            tuple(ey_m1) + tuple(ey_0) + tuple(oy_m1) + tuple(oy_0),
            unroll=2,
        )

    def do_pair(b, q, k, prefetch_k, do_wait_out):
        wait_in(b, q)
        if prefetch_k is not None:
            issue_in(1 - b, q, prefetch_k)
        if do_wait_out:
            wait_out(b, q)
        compute(b)
        issue_out(b, q, k)

    for q in range(4):
        issue_in(0, q, k0)
        do_pair(0, q, k0, k0 + 1, False)
        do_pair(1, q, k0 + 1, k0 + 2, False)

        def loop(i, _):
            do_pair(0, q, k0 + 2 * i, k0 + 2 * i + 1, True)
            do_pair(1, q, k0 + 2 * i + 1, k0 + 2 * i + 2, True)
            return 0

        lax.fori_loop(1, _NPAIR // 2 - 1, loop, 0)
        do_pair(0, q, k0 + _NPAIR - 2, k0 + _NPAIR - 1, True)
        do_pair(1, q, k0 + _NPAIR - 1, None, True)
        wait_out(0, q)
        wait_out(1, q)


def kernel(img):
    n, h, w, ch = img.shape
    run = pl.kernel(
        _body,
        out_type=jax.ShapeDtypeStruct((n, 2 * h, 2 * w, ch), jnp.float32),
        mesh=plsc.VectorSubcoreMesh(
            core_axis_name="c", subcore_axis_name="s",
            num_cores=2, num_subcores=16,
        ),
        compiler_params=pltpu.CompilerParams(use_tc_tiling_on_sc=True),
        scratch_types=[
            pltpu.VMEM((2, 3, _IB, _C), jnp.float32),
            pltpu.VMEM((2, 2, 2 * _QW, _C), jnp.float32),
            pltpu.SemaphoreType.DMA,
            pltpu.SemaphoreType.DMA,
            pltpu.SemaphoreType.DMA,
            pltpu.SemaphoreType.DMA,
        ],
    )
    return run(img)


# speedup vs baseline: 63.8980x; 1.4733x over previous
import jax
import jax.numpy as jnp
from jax import lax
from jax.experimental import pallas as pl
from jax.experimental.pallas import tpu as pltpu
from jax.experimental.pallas import tpu_sc as plsc

_H = 224
_W = 224
_NPAIR = 14
_CH = 32
_EYW = 240
_NW = _W // 16


def _body(img, out, inb, outb, eyb, oyb, in_s0, in_s1, out_s0, out_s1):
    batch = lax.axis_index("c")
    seg = lax.axis_index("s")
    k0 = seg * _NPAIR
    in_sems = (in_s0, in_s1)
    out_sems = (out_s0, out_s1)
    lanes = lax.iota(jnp.int32, 16)

    def issue_in(b, hc, k):
        rp = jnp.maximum(k - 1, 0)
        rn = jnp.minimum(k + 1, _H - 1)
        for j, r in enumerate((rp, k, rn)):
            pltpu.async_copy(
                img.at[batch, r, pl.ds(hc * _CH, _CH), :],
                inb.at[b, j],
                in_sems[b],
            )

    def wait_in(b, hc):
        for j in range(3):
            pltpu.make_async_copy(
                img.at[0, 0, pl.ds(hc * _CH, _CH), :],
                inb.at[b, j],
                in_sems[b],
            ).wait()

    def issue_out(b, hc, k):
        for r in range(2):
            pltpu.async_copy(
                outb.at[b, r],
                out.at[batch, 2 * k + r, pl.ds(hc * _CH, _CH), :],
                out_sems[b],
            )

    def wait_out(b, hc):
        for r in range(2):
            pltpu.make_async_copy(
                outb.at[b, r],
                out.at[0, 0, pl.ds(hc * _CH, _CH), :],
                out_sems[b],
            ).wait()

    def compute(b):
        def a_ch(ch, _):
            def a_w(w16, _):
                w0 = w16 * 16
                src = pl.ds(w0, 16)
                pc = inb[b, 0, ch, src]
                cc = inb[b, 1, ch, src]
                nc = inb[b, 2, ch, src]
                dst = pl.ds(ch * _EYW + w0 + 16, 16)
                eyb[dst] = 0.25 * pc + 0.75 * cc
                oyb[dst] = 0.75 * cc + 0.25 * nc
                return 0
            lax.fori_loop(0, _NW, a_w, 0, unroll=2)
            return 0
        lax.fori_loop(0, _CH, a_ch, 0)

        ch16 = lanes * _EYW
        for grp in range(2):
            base = grp * 16 * _EYW
            for ref in (eyb, oyb):
                v0 = plsc.load_gather(ref, [ch16 + (base + 16)])
                plsc.store_scatter(ref, [ch16 + (base + 15)], v0)
                v1 = plsc.load_gather(ref, [ch16 + (base + 239)])
                plsc.store_scatter(ref, [ch16 + (base + 240)], v1)

        ebuf = outb.at[b, 0]
        obuf = outb.at[b, 1]

        def b_ch(ch, _):
            chv = jnp.full((16,), 0, jnp.int32) + ch

            def b_w(w16, _):
                w0 = w16 * 16
                b0 = ch * _EYW + w0 + 16
                ey_m1 = plsc.load_gather(eyb, [lanes + (b0 - 1)])
                ey_0 = eyb[pl.ds(b0, 16)]
                ey_p1 = plsc.load_gather(eyb, [lanes + (b0 + 1)])
                oy_m1 = plsc.load_gather(oyb, [lanes + (b0 - 1)])
                oy_0 = oyb[pl.ds(b0, 16)]
                oy_p1 = plsc.load_gather(oyb, [lanes + (b0 + 1)])
                wev = 2 * lanes + 2 * w0
                wod = wev + 1
                plsc.store_scatter(ebuf, [chv, wev],
                                   0.25 * ey_m1 + 0.75 * ey_0)
                plsc.store_scatter(ebuf, [chv, wod],
                                   0.75 * ey_0 + 0.25 * ey_p1)
                plsc.store_scatter(obuf, [chv, wev],
                                   0.25 * oy_m1 + 0.75 * oy_0)
                plsc.store_scatter(obuf, [chv, wod],
                                   0.75 * oy_0 + 0.25 * oy_p1)
                return 0
            lax.fori_loop(0, _NW, b_w, 0, unroll=2)
            return 0
        lax.fori_loop(0, _CH, b_ch, 0)

    def do_pair(b, hc, k, prefetch_k, do_wait_out):
        wait_in(b, hc)
        if prefetch_k is not None:
            issue_in(1 - b, hc, prefetch_k)
        if do_wait_out:
            wait_out(b, hc)
        compute(b)
        issue_out(b, hc, k)

    for hc in range(2):
        issue_in(0, hc, k0)
        do_pair(0, hc, k0, k0 + 1, False)
        do_pair(1, hc, k0 + 1, k0 + 2, False)

        def loop(i, _):
            do_pair(0, hc, k0 + 2 * i, k0 + 2 * i + 1, True)
            do_pair(1, hc, k0 + 2 * i + 1, k0 + 2 * i + 2, True)
            return 0

        lax.fori_loop(1, _NPAIR // 2 - 1, loop, 0)
        do_pair(0, hc, k0 + _NPAIR - 2, k0 + _NPAIR - 1, True)
        do_pair(1, hc, k0 + _NPAIR - 1, None, True)
        wait_out(0, hc)
        wait_out(1, hc)


def kernel(img):
    n, h, w, ch = img.shape
    img_t = jnp.transpose(img, (0, 1, 3, 2))
    run = pl.kernel(
        _body,
        out_type=jax.ShapeDtypeStruct((n, 2 * h, ch, 2 * w), jnp.float32),
        mesh=plsc.VectorSubcoreMesh(
            core_axis_name="c", subcore_axis_name="s",
            num_cores=2, num_subcores=16,
        ),
        compiler_params=pltpu.CompilerParams(
            use_tc_tiling_on_sc=True, needs_layout_passes=False,
        ),
        scratch_types=[
            pltpu.VMEM((2, 3, _CH, _W), jnp.float32),
            pltpu.VMEM((2, 2, _CH, 2 * _W), jnp.float32),
            pltpu.VMEM((_CH * _EYW + 16,), jnp.float32),
            pltpu.VMEM((_CH * _EYW + 16,), jnp.float32),
            pltpu.SemaphoreType.DMA,
            pltpu.SemaphoreType.DMA,
            pltpu.SemaphoreType.DMA,
            pltpu.SemaphoreType.DMA,
        ],
    )
    out_t = run(img_t)
    return jnp.transpose(out_t, (0, 1, 3, 2))
